# baseline (speedup 1.0000x reference)
import functools

import jax
import jax.numpy as jnp
from jax import lax
from jax.experimental import pallas as pl
from jax.experimental.pallas import tpu as pltpu
from jax.experimental.pallas import tpu_sc as plsc

EMBED_DIM = 64
BATCH = 4096
MAX_LEN = 50
N_TOKENS = BATCH * MAX_LEN

_info = plsc.get_sparse_core_info()
NUM_CORES = _info.num_cores
NUM_SUBCORES = _info.num_subcores
NUM_WORKERS = NUM_CORES * NUM_SUBCORES

T_PER_W = N_TOKENS // NUM_WORKERS
CHUNK = 400
N_CHUNKS = T_PER_W // CHUNK
NBUF = 2


_mesh = plsc.VectorSubcoreMesh(core_axis_name="c", subcore_axis_name="s")


@functools.partial(
    pl.kernel,
    mesh=_mesh,
    out_type=jax.ShapeDtypeStruct((N_TOKENS, EMBED_DIM), jnp.float32),
    scratch_types=[
        pltpu.VMEM((T_PER_W,), jnp.int32),
    ] + [pltpu.VMEM((CHUNK, 128), jnp.float32)] * NBUF
      + [pltpu.SemaphoreType.DMA] * (2 * NBUF),
    compiler_params=pltpu.CompilerParams(use_tc_tiling_on_sc=False),
)
def _gather_kernel(idx_hbm, table_hbm, out_hbm, idx_v, b0, b1, g0, g1, s0, s1):
    bufs = (b0, b1)
    gsem = (g0, g1)
    ssem = (s0, s1)
    wid = lax.axis_index("s") * NUM_CORES + lax.axis_index("c")
    base = wid * T_PER_W
    pltpu.sync_copy(idx_hbm.at[pl.ds(base, T_PER_W)], idx_v)

    gcp = [None] * N_CHUNKS
    scp = [None] * N_CHUNKS

    def start_gather(ci):
        b = ci % NBUF
        gcp[ci] = pltpu.async_copy(
            table_hbm.at[idx_v.at[pl.ds(ci * CHUNK, CHUNK)]], bufs[b], gsem[b]
        )

    for ci in range(min(NBUF, N_CHUNKS)):
        start_gather(ci)
    for ci in range(N_CHUNKS):
        b = ci % NBUF
        gcp[ci].wait()
        scp[ci] = pltpu.async_copy(
            bufs[b].at[:, pl.ds(0, EMBED_DIM)],
            out_hbm.at[pl.ds(base + ci * CHUNK, CHUNK)], ssem[b]
        )
        nx = ci + NBUF
        if nx < N_CHUNKS:
            scp[ci].wait()
            start_gather(nx)
    for ci in range(max(0, N_CHUNKS - NBUF), N_CHUNKS):
        scp[ci].wait()


def kernel(inputs, embedding):
    idx = inputs.astype(jnp.int32).reshape(N_TOKENS)
    table = jnp.pad(embedding, ((0, 0), (0, 128 - EMBED_DIM)))
    out = _gather_kernel(idx, table)
    return out.reshape(BATCH, MAX_LEN, EMBED_DIM)


# speedup vs baseline: 1.0020x; 1.0020x over previous
import functools

import jax
import jax.numpy as jnp
from jax import lax
from jax.experimental import pallas as pl
from jax.experimental.pallas import tpu as pltpu
from jax.experimental.pallas import tpu_sc as plsc

EMBED_DIM = 64
BATCH = 4096
MAX_LEN = 50
N_TOKENS = BATCH * MAX_LEN

_info = plsc.get_sparse_core_info()
NUM_CORES = _info.num_cores
NUM_SUBCORES = _info.num_subcores
NUM_WORKERS = NUM_CORES * NUM_SUBCORES

T_PER_W = N_TOKENS // NUM_WORKERS
CHUNK = 320
N_CHUNKS = T_PER_W // CHUNK
NBUF = 2


_mesh = plsc.VectorSubcoreMesh(core_axis_name="c", subcore_axis_name="s")


@functools.partial(
    pl.kernel,
    mesh=_mesh,
    out_type=jax.ShapeDtypeStruct((N_TOKENS, EMBED_DIM), jnp.float32),
    scratch_types=[
        pltpu.VMEM((T_PER_W,), jnp.int32),
    ] + [pltpu.VMEM((CHUNK, 128), jnp.float32)] * NBUF
      + [pltpu.SemaphoreType.DMA] * (2 * NBUF),
    compiler_params=pltpu.CompilerParams(use_tc_tiling_on_sc=False),
)
def _gather_kernel(idx_hbm, table_hbm, out_hbm, idx_v, b0, b1, g0, g1, s0, s1):
    bufs = (b0, b1)
    gsem = (g0, g1)
    ssem = (s0, s1)
    wid = lax.axis_index("s") * NUM_CORES + lax.axis_index("c")
    base = wid * T_PER_W
    pltpu.sync_copy(idx_hbm.at[pl.ds(base, T_PER_W)], idx_v)

    gcp = [None] * N_CHUNKS
    scp = [None] * N_CHUNKS

    def start_gather(ci):
        b = ci % NBUF
        gcp[ci] = pltpu.async_copy(
            table_hbm.at[idx_v.at[pl.ds(ci * CHUNK, CHUNK)]], bufs[b], gsem[b]
        )

    for ci in range(min(NBUF, N_CHUNKS)):
        start_gather(ci)
    for ci in range(N_CHUNKS):
        b = ci % NBUF
        gcp[ci].wait()
        scp[ci] = pltpu.async_copy(
            bufs[b].at[:, pl.ds(0, EMBED_DIM)],
            out_hbm.at[pl.ds(base + ci * CHUNK, CHUNK)], ssem[b]
        )
        nx = ci + NBUF
        if nx < N_CHUNKS:
            scp[ci].wait()
            start_gather(nx)
    for ci in range(max(0, N_CHUNKS - NBUF), N_CHUNKS):
        scp[ci].wait()


def kernel(inputs, embedding):
    idx = inputs.astype(jnp.int32).reshape(N_TOKENS)
    table = jnp.pad(embedding, ((0, 0), (0, 128 - EMBED_DIM)))
    out = _gather_kernel(idx, table)
    return out.reshape(BATCH, MAX_LEN, EMBED_DIM)
